# Initial kernel scaffold; baseline (speedup 1.0000x reference)
import jax
import jax.numpy as jnp
from jax.experimental import pallas as pl


def kernel(x_host, x_user, edge_index_host_auth_host, edge_index_user_authenticates_to_host, proj_host_w1, proj_host_b1, proj_user_w1, proj_user_b1, att_src_hh1, att_dst_hh1, att_src_uh1, att_dst_uh1, k_lin_w1, k_lin_b1, q1, proj_host_w2, proj_host_b2, proj_user_w2, proj_user_b2, att_src_hh2, att_dst_hh2, att_src_uh2, att_dst_uh2, k_lin_w2, k_lin_b2, q2, proj_w, proj_b):
    raise NotImplementedError("write your pallas kernel here")


# speedup vs baseline: 1.2027x; 1.2027x over previous
import functools
import jax
import jax.numpy as jnp
from jax.experimental import pallas as pl
from jax.experimental.pallas import tpu as pltpu


def _proj_block(x_ref, w_ref, b_ref, a_ref, y_ref, al_ref):
    y = jnp.dot(x_ref[...], w_ref[...], preferred_element_type=jnp.float32)
    y = y + b_ref[...]
    y_ref[...] = y
    al_ref[...] = jnp.dot(y, a_ref[...], preferred_element_type=jnp.float32)


def _project(x, w, b, A, block=1000):
    n, din = x.shape
    dout = w.shape[1]
    k = A.shape[1]
    return pl.pallas_call(
        _proj_block,
        grid=(n // block,),
        in_specs=[
            pl.BlockSpec((block, din), lambda i: (i, 0)),
            pl.BlockSpec((din, dout), lambda i: (0, 0)),
            pl.BlockSpec((dout,), lambda i: (0,)),
            pl.BlockSpec((dout, k), lambda i: (0, 0)),
        ],
        out_specs=[
            pl.BlockSpec((block, dout), lambda i: (i, 0)),
            pl.BlockSpec((block, k), lambda i: (i, 0)),
        ],
        out_shape=[
            jax.ShapeDtypeStruct((n, dout), jnp.float32),
            jax.ShapeDtypeStruct((n, k), jnp.float32),
        ],
    )(x, w, b, A)


def _att_mat(a):
    heads, d = a.shape
    hsel = jnp.repeat(jnp.eye(heads, dtype=a.dtype), d, axis=0)
    return hsel * a.reshape(-1)[:, None]


def _agg(y_src, als, ald, ei, heads, d, n_dst):
    src, dst = ei[0], ei[1]
    al = als[src] + ald[dst]
    al = jnp.where(al >= 0, al, 0.2 * al)
    w = jnp.exp(al)
    ssum = jax.ops.segment_sum(w, dst, num_segments=n_dst)
    msg = y_src.reshape(-1, heads, d)[src] * w[:, :, None]
    acc = jax.ops.segment_sum(msg, dst, num_segments=n_dst)
    out = acc / (ssum[:, :, None] + 1e-16)
    return out.reshape(n_dst, heads * d), ssum


def kernel(x_host, x_user, edge_index_host_auth_host, edge_index_user_authenticates_to_host,
           proj_host_w1, proj_host_b1, proj_user_w1, proj_user_b1,
           att_src_hh1, att_dst_hh1, att_src_uh1, att_dst_uh1, k_lin_w1, k_lin_b1, q1,
           proj_host_w2, proj_host_b2, proj_user_w2, proj_user_b2,
           att_src_hh2, att_dst_hh2, att_src_uh2, att_dst_uh2, k_lin_w2, k_lin_b2, q2,
           proj_w, proj_b):
    ei_hh = edge_index_host_auth_host
    ei_uh = edge_index_user_authenticates_to_host
    n = x_host.shape[0]

    A_host1 = jnp.concatenate(
        [_att_mat(att_src_hh1), _att_mat(att_dst_hh1), _att_mat(att_dst_uh1)], axis=1)
    yh, al_h = _project(x_host, proj_host_w1, proj_host_b1, A_host1)
    als_hh, ald_hh, ald_uh = al_h[:, 0:4], al_h[:, 4:8], al_h[:, 8:12]
    yu, als_uh = _project(x_user, proj_user_w1, proj_user_b1, _att_mat(att_src_uh1))

    o_hh, _ = _agg(yh, als_hh, ald_hh, ei_hh, 4, 16, n)
    o_uh, ssum_uh = _agg(yu, als_uh, ald_uh, ei_uh, 4, 16, n)
    o_hh = jax.nn.relu(o_hh)
    o_uh = jax.nn.relu(o_uh)

    stk = jnp.stack([o_hh, o_uh])
    kmat = jnp.tanh(stk @ k_lin_w1 + k_lin_b1).mean(axis=1)
    score = (q1[None, :] * kmat).sum(-1)
    attn = jax.nn.softmax(score)
    h_host = jax.nn.relu(attn[0] * o_hh + attn[1] * o_uh)

    A_host2 = jnp.concatenate([_att_mat(att_src_hh2), _att_mat(att_dst_hh2)], axis=1)
    yh2, al2 = _project(h_host, proj_host_w2, proj_host_b2, A_host2)
    o_hh2, _ = _agg(yh2, al2[:, 0:1], al2[:, 1:2], ei_hh, 1, 64, n)
    o_hh2 = jax.nn.relu(o_hh2)

    covered = (ssum_uh[:, 0] > 0).astype(jnp.float32)
    frac = covered.sum() / n
    v1 = jax.nn.relu(proj_user_b2)

    kmat_hh2 = jnp.tanh(o_hh2 @ k_lin_w2 + k_lin_b2).mean(axis=0)
    kmat_uh2 = frac * jnp.tanh(v1 @ k_lin_w2 + k_lin_b2) + (1 - frac) * jnp.tanh(k_lin_b2)
    score2 = jnp.stack([(q2 * kmat_hh2).sum(), (q2 * kmat_uh2).sum()])
    attn2 = jax.nn.softmax(score2)
    emb = attn2[0] * o_hh2.mean(axis=0) + attn2[1] * (frac * v1)
    return emb @ proj_w + proj_b


# speedup vs baseline: 60.5022x; 50.3049x over previous
import functools
import jax
import jax.numpy as jnp
from jax import lax
from jax.experimental import pallas as pl
from jax.experimental.pallas import tpu as pltpu
from jax.experimental.pallas import tpu_sc as plsc

N_HOST = 50000
N_USER = 25000
NPAD = 50048
ROWS_PER_TILE = NPAD // 16
C = 2


def _proj_block(x_ref, w_ref, b_ref, a_ref, y_ref, al_ref):
    y = jnp.dot(x_ref[...], w_ref[...], preferred_element_type=jnp.float32)
    y = y + b_ref[...]
    y_ref[...] = y
    al_ref[...] = jnp.dot(y, a_ref[...], preferred_element_type=jnp.float32)


def _project(x, w, b, A, block=1000):
    n, din = x.shape
    dout = w.shape[1]
    k = A.shape[1]
    return pl.pallas_call(
        _proj_block,
        grid=(n // block,),
        in_specs=[
            pl.BlockSpec((block, din), lambda i: (i, 0)),
            pl.BlockSpec((din, dout), lambda i: (0, 0)),
            pl.BlockSpec((dout,), lambda i: (0,)),
            pl.BlockSpec((dout, k), lambda i: (0, 0)),
        ],
        out_specs=[
            pl.BlockSpec((block, dout), lambda i: (i, 0)),
            pl.BlockSpec((block, k), lambda i: (i, 0)),
        ],
        out_shape=[
            jax.ShapeDtypeStruct((n, dout), jnp.float32),
            jax.ShapeDtypeStruct((n, k), jnp.float32),
        ],
    )(x, w, b, A)


def _att_mat(a):
    heads, d = a.shape
    hsel = jnp.repeat(jnp.eye(heads, dtype=a.dtype), d, axis=0)
    return hsel * a.reshape(-1)[:, None]


def _make_sc_agg(n_edges_pad, n_src, subh):
    rows = n_edges_pad // 128
    rows_per_tile = rows // 16
    nblocks = rows_per_tile // C
    ssw = ROWS_PER_TILE * subh

    def body(srcw, dstw, x2, alsA, alsB, aldA, aldB, acc_o, ssum_o,
             sidx, didx, doff, di0, di1, xb, a0, a1, d0, d1, zba, zbs,
             acc_s, ssum_s, sem0, sem1, sem2, sem3, sem4):
        sc = lax.axis_index("c")
        tid = lax.axis_index("s")
        zv = jnp.zeros((16,), jnp.float32)

        @pl.loop(0, 136)
        def _(r):
            zba[r, pl.ds(0, 16)] = zv
            zba[r, pl.ds(16, 16)] = zv

        @pl.loop(0, 23)
        def _(i):
            zbs[pl.ds(i * 16, 16)] = zv

        r0 = tid * ROWS_PER_TILE
        @pl.loop(0, 23)
        def _(q):
            pltpu.sync_copy(zba, acc_s.at[pl.ds(r0 + q * 136, 136), :])
        zchunk = 368 if subh == 2 else 136
        nz = ssw // zchunk
        @pl.loop(0, nz)
        def _(q):
            pltpu.sync_copy(zbs.at[pl.ds(0, zchunk)],
                            ssum_s.at[pl.ds(tid * ssw + q * zchunk, zchunk)])
        plsc.subcore_barrier()

        src_off = sc * n_src
        dst_off = sc * NPAD
        erow0 = tid * rows_per_tile

        @pl.loop(0, nblocks)
        def _(b):
            rb = erow0 + b * C
            pltpu.sync_copy(srcw.at[pl.ds(rb, C), :], sidx)
            pltpu.sync_copy(dstw.at[pl.ds(rb, C), :], didx)
            for j in range(C):
                @pl.loop(0, 8)
                def _(cc):
                    s = sidx[j, pl.ds(cc * 16, 16)]
                    sidx[j, pl.ds(cc * 16, 16)] = s + src_off
                    d = didx[j, pl.ds(cc * 16, 16)]
                    doff[j, pl.ds(cc * 16, 16)] = d + dst_off
                    if subh == 2:
                        di0[j, pl.ds(cc * 16, 16)] = d * 2
                        di1[j, pl.ds(cc * 16, 16)] = d * 2 + 1
                    else:
                        di0[j, pl.ds(cc * 16, 16)] = d

            cps = []
            for j in range(C):
                cps.append(pltpu.async_copy(x2.at[sidx.at[j]], xb.at[j], sem0))
                cps.append(pltpu.async_copy(alsA.at[sidx.at[j]], a0.at[j], sem1))
                cps.append(pltpu.async_copy(aldA.at[doff.at[j]], d0.at[j], sem3))
                if subh == 2:
                    cps.append(pltpu.async_copy(alsB.at[sidx.at[j]], a1.at[j], sem2))
                    cps.append(pltpu.async_copy(aldB.at[doff.at[j]], d1.at[j], sem4))
            for cp in cps:
                cp.wait()

            for j in range(C):
                @pl.loop(0, 8)
                def _(cc):
                    al0 = a0[j, pl.ds(cc * 16, 16)] + d0[j, pl.ds(cc * 16, 16)]
                    a0[j, pl.ds(cc * 16, 16)] = jnp.exp(
                        jnp.where(al0 >= 0, al0, 0.2 * al0))
                    if subh == 2:
                        al1 = a1[j, pl.ds(cc * 16, 16)] + d1[j, pl.ds(cc * 16, 16)]
                        a1[j, pl.ds(cc * 16, 16)] = jnp.exp(
                            jnp.where(al1 >= 0, al1, 0.2 * al1))

            for j in range(C):
                @pl.loop(0, 8)
                def _(cc):
                    wv0 = a0[j, pl.ds(cc * 16, 16)]
                    wv1 = a1[j, pl.ds(cc * 16, 16)] if subh == 2 else wv0
                    for l in range(16):
                        e = cc * 16 + l
                        xb[j, e, pl.ds(0, 16)] = xb[j, e, pl.ds(0, 16)] * wv0[l]
                        xb[j, e, pl.ds(16, 16)] = xb[j, e, pl.ds(16, 16)] * wv1[l]

            for j in range(C):
                pltpu.sync_copy(xb.at[j], acc_s.at[didx.at[j]], add=True)
                pltpu.sync_copy(a0.at[j], ssum_s.at[di0.at[j]], add=True)
                if subh == 2:
                    pltpu.sync_copy(a1.at[j], ssum_s.at[di1.at[j]], add=True)

        plsc.subcore_barrier()
        pltpu.sync_copy(acc_s.at[pl.ds(r0, ROWS_PER_TILE), :],
                        acc_o.at[pl.ds(dst_off + r0, ROWS_PER_TILE), :])
        pltpu.sync_copy(ssum_s.at[pl.ds(tid * ssw, ssw)],
                        ssum_o.at[pl.ds(sc * NPAD * subh + tid * ssw, ssw)])

    mesh = plsc.VectorSubcoreMesh(core_axis_name="c", subcore_axis_name="s")
    f32, i32 = jnp.float32, jnp.int32
    return pl.kernel(
        body,
        out_type=[jax.ShapeDtypeStruct((2 * NPAD, 32), f32),
                  jax.ShapeDtypeStruct((2 * NPAD * subh,), f32)],
        mesh=mesh,
        compiler_params=pltpu.CompilerParams(use_tc_tiling_on_sc=False),
        scratch_types=[
            pltpu.VMEM((C, 128), i32),
            pltpu.VMEM((C, 128), i32),
            pltpu.VMEM((C, 128), i32),
            pltpu.VMEM((C, 128), i32),
            pltpu.VMEM((C, 128), i32),
            pltpu.VMEM((C, 128, 32), f32),
            pltpu.VMEM((C, 128), f32),
            pltpu.VMEM((C, 128), f32),
            pltpu.VMEM((C, 128), f32),
            pltpu.VMEM((C, 128), f32),
            pltpu.VMEM((136, 32), f32),
            pltpu.VMEM((368,), f32),
            pltpu.VMEM_SHARED((NPAD, 32), f32),
            pltpu.VMEM_SHARED((NPAD * subh,), f32),
            pltpu.SemaphoreType.DMA,
            pltpu.SemaphoreType.DMA,
            pltpu.SemaphoreType.DMA,
            pltpu.SemaphoreType.DMA,
            pltpu.SemaphoreType.DMA,
        ],
    )


def _pad_edges(src, dst, n_src, n_edges_pad):
    e = src.shape[0]
    npad = n_edges_pad - e
    if npad:
        ar = jnp.arange(npad, dtype=jnp.int32)
        src = jnp.concatenate([src, ar % n_src])
        dst = jnp.concatenate([dst, N_HOST + ar % 48])
    return src.reshape(-1, 128), dst.reshape(-1, 128)


def _split2(x):
    return jnp.concatenate([x[:, :32], x[:, 32:]], axis=0)


def _pad_dst_tab(col_a, col_b):
    z = jnp.zeros((NPAD - N_HOST,), jnp.float32)
    return jnp.concatenate([col_a, z, col_b, z])


def _merge(acc, ssum, subh):
    o = jnp.concatenate([acc[:N_HOST], acc[NPAD:NPAD + N_HOST]], axis=1)
    ss = ssum.reshape(2, NPAD, subh)
    s = jnp.concatenate([ss[0, :N_HOST], ss[1, :N_HOST]], axis=1)
    return o, s


def kernel(x_host, x_user, edge_index_host_auth_host, edge_index_user_authenticates_to_host,
           proj_host_w1, proj_host_b1, proj_user_w1, proj_user_b1,
           att_src_hh1, att_dst_hh1, att_src_uh1, att_dst_uh1, k_lin_w1, k_lin_b1, q1,
           proj_host_w2, proj_host_b2, proj_user_w2, proj_user_b2,
           att_src_hh2, att_dst_hh2, att_src_uh2, att_dst_uh2, k_lin_w2, k_lin_b2, q2,
           proj_w, proj_b):
    ei_hh = edge_index_host_auth_host
    ei_uh = edge_index_user_authenticates_to_host
    n = N_HOST

    E_HH_PAD = 802816
    E_UH_PAD = 409600
    src_hh, dst_hh = _pad_edges(ei_hh[0], ei_hh[1], N_HOST, E_HH_PAD)
    src_uh, dst_uh = _pad_edges(ei_uh[0], ei_uh[1], N_USER, E_UH_PAD)

    agg_hh = _make_sc_agg(E_HH_PAD, N_HOST, 2)
    agg_uh = _make_sc_agg(E_UH_PAD, N_USER, 2)
    agg_hh2 = _make_sc_agg(E_HH_PAD, N_HOST, 1)

    A_host1 = jnp.concatenate(
        [_att_mat(att_src_hh1), _att_mat(att_dst_hh1), _att_mat(att_dst_uh1)], axis=1)
    yh, al_h = _project(x_host, proj_host_w1, proj_host_b1, A_host1)
    yu, als_uh = _project(x_user, proj_user_w1, proj_user_b1, _att_mat(att_src_uh1))

    acc, ss = agg_hh(src_hh, dst_hh, _split2(yh),
                     jnp.concatenate([al_h[:, 0], al_h[:, 2]]),
                     jnp.concatenate([al_h[:, 1], al_h[:, 3]]),
                     _pad_dst_tab(al_h[:, 4], al_h[:, 6]),
                     _pad_dst_tab(al_h[:, 5], al_h[:, 7]))
    o_hh, s_hh = _merge(acc, ss, 2)

    acc, ss = agg_uh(src_uh, dst_uh, _split2(yu),
                     jnp.concatenate([als_uh[:, 0], als_uh[:, 2]]),
                     jnp.concatenate([als_uh[:, 1], als_uh[:, 3]]),
                     _pad_dst_tab(al_h[:, 8], al_h[:, 10]),
                     _pad_dst_tab(al_h[:, 9], al_h[:, 11]))
    o_uh, s_uh = _merge(acc, ss, 2)

    o_hh = jax.nn.relu(o_hh / (jnp.repeat(s_hh, 16, axis=1) + 1e-16))
    o_uh = jax.nn.relu(o_uh / (jnp.repeat(s_uh, 16, axis=1) + 1e-16))

    stk = jnp.stack([o_hh, o_uh])
    kmat = jnp.tanh(stk @ k_lin_w1 + k_lin_b1).mean(axis=1)
    score = (q1[None, :] * kmat).sum(-1)
    attn = jax.nn.softmax(score)
    h_host = jax.nn.relu(attn[0] * o_hh + attn[1] * o_uh)

    A_host2 = jnp.concatenate([_att_mat(att_src_hh2), _att_mat(att_dst_hh2)], axis=1)
    yh2, al2 = _project(h_host, proj_host_w2, proj_host_b2, A_host2)
    acc, ss = agg_hh2(src_hh, dst_hh, _split2(yh2),
                      jnp.concatenate([al2[:, 0], al2[:, 0]]),
                      jnp.concatenate([al2[:, 0], al2[:, 0]]),
                      _pad_dst_tab(al2[:, 1], al2[:, 1]),
                      _pad_dst_tab(al2[:, 1], al2[:, 1]))
    o2, s2 = _merge(acc, ss, 1)
    o_hh2 = jax.nn.relu(o2 / (s2[:, 0:1] + 1e-16))

    covered = (s_uh[:, 0] > 0).astype(jnp.float32)
    frac = covered.sum() / n
    v1 = jax.nn.relu(proj_user_b2)

    kmat_hh2 = jnp.tanh(o_hh2 @ k_lin_w2 + k_lin_b2).mean(axis=0)
    kmat_uh2 = frac * jnp.tanh(v1 @ k_lin_w2 + k_lin_b2) + (1 - frac) * jnp.tanh(k_lin_b2)
    score2 = jnp.stack([(q2 * kmat_hh2).sum(), (q2 * kmat_uh2).sum()])
    attn2 = jax.nn.softmax(score2)
    emb = attn2[0] * o_hh2.mean(axis=0) + attn2[1] * (frac * v1)
    return emb @ proj_w + proj_b


# speedup vs baseline: 78.6415x; 1.2998x over previous
import functools
import jax
import jax.numpy as jnp
from jax import lax
from jax.experimental import pallas as pl
from jax.experimental.pallas import tpu as pltpu
from jax.experimental.pallas import tpu_sc as plsc

N_HOST = 50000
N_USER = 25000
NPAD = 50048
ROWS_PER_TILE = NPAD // 16
C = 2


def _proj_block(x_ref, w_ref, b_ref, a_ref, y_ref, al_ref):
    y = jnp.dot(x_ref[...], w_ref[...], preferred_element_type=jnp.float32)
    y = y + b_ref[...]
    y_ref[...] = y
    al_ref[...] = jnp.dot(y, a_ref[...], preferred_element_type=jnp.float32)


def _project(x, w, b, A, block=1000):
    n, din = x.shape
    dout = w.shape[1]
    k = A.shape[1]
    return pl.pallas_call(
        _proj_block,
        grid=(n // block,),
        in_specs=[
            pl.BlockSpec((block, din), lambda i: (i, 0)),
            pl.BlockSpec((din, dout), lambda i: (0, 0)),
            pl.BlockSpec((dout,), lambda i: (0,)),
            pl.BlockSpec((dout, k), lambda i: (0, 0)),
        ],
        out_specs=[
            pl.BlockSpec((block, dout), lambda i: (i, 0)),
            pl.BlockSpec((block, k), lambda i: (i, 0)),
        ],
        out_shape=[
            jax.ShapeDtypeStruct((n, dout), jnp.float32),
            jax.ShapeDtypeStruct((n, k), jnp.float32),
        ],
    )(x, w, b, A)


def _att_mat(a):
    heads, d = a.shape
    hsel = jnp.repeat(jnp.eye(heads, dtype=a.dtype), d, axis=0)
    return hsel * a.reshape(-1)[:, None]


def _make_sc_agg(n_edges_pad, n_src, subh):
    rows = n_edges_pad // 128
    rows_per_tile = rows // 16
    npairs = rows_per_tile // (2 * C)
    ssw = ROWS_PER_TILE * subh

    def body(srcw, dstw, x2, alsA, alsB, aldA, aldB, acc_o, ssum_o,
             sxc, dxc,
             soff0, soff1, doff0, doff1, di00, di01, di10, di11,
             xb0, xb1, a00, a01, a10, a11, d00, d01, d10, d11,
             zba, zbs, acc_s, ssum_s, semg0, semg1, sems0, sems1):
        soff = [soff0, soff1]
        doff = [doff0, doff1]
        di0 = [di00, di01]
        di1 = [di10, di11]
        xb = [xb0, xb1]
        a0 = [a00, a01]
        a1 = [a10, a11]
        d0 = [d00, d01]
        d1 = [d10, d11]
        semg = [semg0, semg1]
        sems = [sems0, sems1]

        sc = lax.axis_index("c")
        tid = lax.axis_index("s")
        zv = jnp.zeros((16,), jnp.float32)

        @pl.loop(0, 46)
        def _(r):
            zba[r, pl.ds(0, 16)] = zv
            zba[r, pl.ds(16, 16)] = zv

        @pl.loop(0, 23)
        def _(i):
            zbs[pl.ds(i * 16, 16)] = zv

        r0 = tid * ROWS_PER_TILE
        @pl.loop(0, 68)
        def _(q):
            pltpu.sync_copy(zba, acc_s.at[pl.ds(r0 + q * 46, 46), :])
        zchunk = 368 if subh == 2 else 136
        @pl.loop(0, ssw // zchunk)
        def _(q):
            pltpu.sync_copy(zbs.at[pl.ds(0, zchunk)],
                            ssum_s.at[pl.ds(tid * ssw + q * zchunk, zchunk)])
        plsc.subcore_barrier()

        src_off = sc * n_src
        dst_off = sc * NPAD
        erow0 = tid * rows_per_tile

        def refill(rowc):
            pltpu.sync_copy(srcw.at[pl.ds(rowc, 4 * C), :], sxc)
            pltpu.sync_copy(dstw.at[pl.ds(rowc, 4 * C), :], dxc)

        def derive(p, lo):
            for j in range(C):
                @pl.loop(0, 8)
                def _(cc):
                    s = sxc[lo + j, pl.ds(cc * 16, 16)]
                    soff[p][j, pl.ds(cc * 16, 16)] = s + src_off
                    d = dxc[lo + j, pl.ds(cc * 16, 16)]
                    doff[p][j, pl.ds(cc * 16, 16)] = d + dst_off
                    if subh == 2:
                        di0[p][j, pl.ds(cc * 16, 16)] = d * 2
                        di1[p][j, pl.ds(cc * 16, 16)] = d * 2 + 1
                    else:
                        di0[p][j, pl.ds(cc * 16, 16)] = d

        def gather_pairs(p):
            prs = []
            for j in range(C):
                prs.append((x2.at[soff[p].at[j]], xb[p].at[j]))
                prs.append((alsA.at[soff[p].at[j]], a0[p].at[j]))
                prs.append((aldA.at[doff[p].at[j]], d0[p].at[j]))
                if subh == 2:
                    prs.append((alsB.at[soff[p].at[j]], a1[p].at[j]))
                    prs.append((aldB.at[doff[p].at[j]], d1[p].at[j]))
            return prs

        def issue_gathers(p):
            for s_, t_ in gather_pairs(p):
                pltpu.async_copy(s_, t_, semg[p])

        def wait_gathers(p):
            for s_, t_ in gather_pairs(p):
                pltpu.make_async_copy(s_, t_, semg[p]).wait()

        def compute(p):
            for j in range(C):
                @pl.loop(0, 8)
                def _(cc):
                    al0 = a0[p][j, pl.ds(cc * 16, 16)] + d0[p][j, pl.ds(cc * 16, 16)]
                    a0[p][j, pl.ds(cc * 16, 16)] = jnp.exp(
                        jnp.where(al0 >= 0, al0, 0.2 * al0))
                    if subh == 2:
                        al1 = a1[p][j, pl.ds(cc * 16, 16)] + d1[p][j, pl.ds(cc * 16, 16)]
                        a1[p][j, pl.ds(cc * 16, 16)] = jnp.exp(
                            jnp.where(al1 >= 0, al1, 0.2 * al1))
            for j in range(C):
                @pl.loop(0, 8)
                def _(cc):
                    wv0 = a0[p][j, pl.ds(cc * 16, 16)]
                    wv1 = a1[p][j, pl.ds(cc * 16, 16)] if subh == 2 else wv0
                    for l in range(16):
                        e = cc * 16 + l
                        xb[p][j, e, pl.ds(0, 16)] = xb[p][j, e, pl.ds(0, 16)] * wv0[l]
                        xb[p][j, e, pl.ds(16, 16)] = xb[p][j, e, pl.ds(16, 16)] * wv1[l]

        def issue_scatters(p, lo):
            descs = []
            for j in range(C):
                descs.append(pltpu.async_copy(
                    xb[p].at[j], acc_s.at[dxc.at[lo + j]], sems[p], add=True))
                descs.append(pltpu.async_copy(
                    a0[p].at[j], ssum_s.at[di0[p].at[j]], sems[p], add=True))
                if subh == 2:
                    descs.append(pltpu.async_copy(
                        a1[p].at[j], ssum_s.at[di1[p].at[j]], sems[p], add=True))
            return descs

        refill(erow0)
        derive(0, 0)
        issue_gathers(0)
        derive(1, C)
        issue_gathers(1)

        @pl.loop(0, npairs)
        def _(g):
            lo = (g % 2) * (2 * C)
            wait_gathers(0)
            compute(0)
            sc0 = issue_scatters(0, lo)
            wait_gathers(1)
            compute(1)
            sc1 = issue_scatters(1, lo + C)
            for dsc in sc0:
                dsc.wait()
            for dsc in sc1:
                dsc.wait()
            gn = g + 1

            @pl.when(jnp.logical_and(gn % 2 == 0, gn < npairs))
            def _():
                refill(erow0 + 2 * C * gn)

            @pl.when(gn < npairs)
            def _():
                nlo = (gn % 2) * (2 * C)
                derive(0, nlo)
                issue_gathers(0)
                derive(1, nlo + C)
                issue_gathers(1)

        plsc.subcore_barrier()
        pltpu.sync_copy(acc_s.at[pl.ds(r0, ROWS_PER_TILE), :],
                        acc_o.at[pl.ds(dst_off + r0, ROWS_PER_TILE), :])
        pltpu.sync_copy(ssum_s.at[pl.ds(tid * ssw, ssw)],
                        ssum_o.at[pl.ds(sc * NPAD * subh + tid * ssw, ssw)])

    mesh = plsc.VectorSubcoreMesh(core_axis_name="c", subcore_axis_name="s")
    f32, i32 = jnp.float32, jnp.int32
    idxb = pltpu.VMEM((C, 128), i32)
    fb = pltpu.VMEM((C, 128), f32)
    return pl.kernel(
        body,
        out_type=[jax.ShapeDtypeStruct((2 * NPAD, 32), f32),
                  jax.ShapeDtypeStruct((2 * NPAD * subh,), f32)],
        mesh=mesh,
        compiler_params=pltpu.CompilerParams(use_tc_tiling_on_sc=False),
        scratch_types=[
            pltpu.VMEM((4 * C, 128), i32),
            pltpu.VMEM((4 * C, 128), i32),
            idxb, idxb,
            idxb, idxb,
            idxb, idxb,
            idxb, idxb,
            pltpu.VMEM((C, 128, 32), f32),
            pltpu.VMEM((C, 128, 32), f32),
            fb, fb, fb, fb,
            fb, fb, fb, fb,
            pltpu.VMEM((46, 32), f32),
            pltpu.VMEM((368,), f32),
            pltpu.VMEM_SHARED((NPAD, 32), f32),
            pltpu.VMEM_SHARED((NPAD * subh,), f32),
            pltpu.SemaphoreType.DMA,
            pltpu.SemaphoreType.DMA,
            pltpu.SemaphoreType.DMA,
            pltpu.SemaphoreType.DMA,
        ],
    )


def _pad_edges(src, dst, n_src, n_edges_pad):
    e = src.shape[0]
    npad = n_edges_pad - e
    if npad:
        ar = jnp.arange(npad, dtype=jnp.int32)
        src = jnp.concatenate([src, ar % n_src])
        dst = jnp.concatenate([dst, N_HOST + ar % 48])
    return src.reshape(-1, 128), dst.reshape(-1, 128)


def _split2(x):
    return jnp.concatenate([x[:, :32], x[:, 32:]], axis=0)


def _pad_dst_tab(col_a, col_b):
    z = jnp.zeros((NPAD - N_HOST,), jnp.float32)
    return jnp.concatenate([col_a, z, col_b, z])


def _merge(acc, ssum, subh):
    o = jnp.concatenate([acc[:N_HOST], acc[NPAD:NPAD + N_HOST]], axis=1)
    ss = ssum.reshape(2, NPAD, subh)
    s = jnp.concatenate([ss[0, :N_HOST], ss[1, :N_HOST]], axis=1)
    return o, s


def kernel(x_host, x_user, edge_index_host_auth_host, edge_index_user_authenticates_to_host,
           proj_host_w1, proj_host_b1, proj_user_w1, proj_user_b1,
           att_src_hh1, att_dst_hh1, att_src_uh1, att_dst_uh1, k_lin_w1, k_lin_b1, q1,
           proj_host_w2, proj_host_b2, proj_user_w2, proj_user_b2,
           att_src_hh2, att_dst_hh2, att_src_uh2, att_dst_uh2, k_lin_w2, k_lin_b2, q2,
           proj_w, proj_b):
    ei_hh = edge_index_host_auth_host
    ei_uh = edge_index_user_authenticates_to_host
    n = N_HOST

    E_HH_PAD = 802816
    E_UH_PAD = 409600
    src_hh, dst_hh = _pad_edges(ei_hh[0], ei_hh[1], N_HOST, E_HH_PAD)
    src_uh, dst_uh = _pad_edges(ei_uh[0], ei_uh[1], N_USER, E_UH_PAD)

    agg_hh = _make_sc_agg(E_HH_PAD, N_HOST, 2)
    agg_uh = _make_sc_agg(E_UH_PAD, N_USER, 2)
    agg_hh2 = _make_sc_agg(E_HH_PAD, N_HOST, 1)

    A_host1 = jnp.concatenate(
        [_att_mat(att_src_hh1), _att_mat(att_dst_hh1), _att_mat(att_dst_uh1)], axis=1)
    yh, al_h = _project(x_host, proj_host_w1, proj_host_b1, A_host1)
    yu, als_uh = _project(x_user, proj_user_w1, proj_user_b1, _att_mat(att_src_uh1))

    acc, ss = agg_hh(src_hh, dst_hh, _split2(yh),
                     jnp.concatenate([al_h[:, 0], al_h[:, 2]]),
                     jnp.concatenate([al_h[:, 1], al_h[:, 3]]),
                     _pad_dst_tab(al_h[:, 4], al_h[:, 6]),
                     _pad_dst_tab(al_h[:, 5], al_h[:, 7]))
    o_hh, s_hh = _merge(acc, ss, 2)

    acc, ss = agg_uh(src_uh, dst_uh, _split2(yu),
                     jnp.concatenate([als_uh[:, 0], als_uh[:, 2]]),
                     jnp.concatenate([als_uh[:, 1], als_uh[:, 3]]),
                     _pad_dst_tab(al_h[:, 8], al_h[:, 10]),
                     _pad_dst_tab(al_h[:, 9], al_h[:, 11]))
    o_uh, s_uh = _merge(acc, ss, 2)

    o_hh = jax.nn.relu(o_hh / (jnp.repeat(s_hh, 16, axis=1) + 1e-16))
    o_uh = jax.nn.relu(o_uh / (jnp.repeat(s_uh, 16, axis=1) + 1e-16))

    stk = jnp.stack([o_hh, o_uh])
    kmat = jnp.tanh(stk @ k_lin_w1 + k_lin_b1).mean(axis=1)
    score = (q1[None, :] * kmat).sum(-1)
    attn = jax.nn.softmax(score)
    h_host = jax.nn.relu(attn[0] * o_hh + attn[1] * o_uh)

    A_host2 = jnp.concatenate([_att_mat(att_src_hh2), _att_mat(att_dst_hh2)], axis=1)
    yh2, al2 = _project(h_host, proj_host_w2, proj_host_b2, A_host2)
    acc, ss = agg_hh2(src_hh, dst_hh, _split2(yh2),
                      jnp.concatenate([al2[:, 0], al2[:, 0]]),
                      jnp.concatenate([al2[:, 0], al2[:, 0]]),
                      _pad_dst_tab(al2[:, 1], al2[:, 1]),
                      _pad_dst_tab(al2[:, 1], al2[:, 1]))
    o2, s2 = _merge(acc, ss, 1)
    o_hh2 = jax.nn.relu(o2 / (s2[:, 0:1] + 1e-16))

    covered = (s_uh[:, 0] > 0).astype(jnp.float32)
    frac = covered.sum() / n
    v1 = jax.nn.relu(proj_user_b2)

    kmat_hh2 = jnp.tanh(o_hh2 @ k_lin_w2 + k_lin_b2).mean(axis=0)
    kmat_uh2 = frac * jnp.tanh(v1 @ k_lin_w2 + k_lin_b2) + (1 - frac) * jnp.tanh(k_lin_b2)
    score2 = jnp.stack([(q2 * kmat_hh2).sum(), (q2 * kmat_uh2).sum()])
    attn2 = jax.nn.softmax(score2)
    emb = attn2[0] * o_hh2.mean(axis=0) + attn2[1] * (frac * v1)
    return emb @ proj_w + proj_b


# speedup vs baseline: 87.9441x; 1.1183x over previous
import functools
import jax
import jax.numpy as jnp
from jax import lax
from jax.experimental import pallas as pl
from jax.experimental.pallas import tpu as pltpu
from jax.experimental.pallas import tpu_sc as plsc

N_HOST = 50000
N_USER = 25000
NPAD = 50048
ROWS_PER_TILE = NPAD // 16
C = 2


def _proj_block(x_ref, w_ref, b_ref, a_ref, y_ref, al_ref):
    y = jnp.dot(x_ref[...], w_ref[...], preferred_element_type=jnp.float32)
    y = y + b_ref[...]
    y_ref[...] = y
    al_ref[...] = jnp.dot(y, a_ref[...], preferred_element_type=jnp.float32)


def _project(x, w, b, A, block=1000):
    n, din = x.shape
    dout = w.shape[1]
    k = A.shape[1]
    return pl.pallas_call(
        _proj_block,
        grid=(n // block,),
        in_specs=[
            pl.BlockSpec((block, din), lambda i: (i, 0)),
            pl.BlockSpec((din, dout), lambda i: (0, 0)),
            pl.BlockSpec((dout,), lambda i: (0,)),
            pl.BlockSpec((dout, k), lambda i: (0, 0)),
        ],
        out_specs=[
            pl.BlockSpec((block, dout), lambda i: (i, 0)),
            pl.BlockSpec((block, k), lambda i: (i, 0)),
        ],
        out_shape=[
            jax.ShapeDtypeStruct((n, dout), jnp.float32),
            jax.ShapeDtypeStruct((n, k), jnp.float32),
        ],
    )(x, w, b, A)


def _att_mat(a):
    heads, d = a.shape
    hsel = jnp.repeat(jnp.eye(heads, dtype=a.dtype), d, axis=0)
    return hsel * a.reshape(-1)[:, None]


def _make_sc_agg(n_edges_pad, n_src, subh):
    rows = n_edges_pad // 128
    rows_per_tile = rows // 16
    npairs = rows_per_tile // (2 * C)
    ssw = ROWS_PER_TILE * subh

    def body(srcw, dstw, x2, alsA, alsB, aldA, aldB, acc_o, ssum_o,
             sxc, dxc,
             soff0, soff1, doff0, doff1, di00, di01, di10, di11,
             xb0, xb1, a00, a01, a10, a11, d00, d01, d10, d11,
             zba, zbs, acc_s, ssum_s, semg0, semg1, sems0, sems1):
        soff = [soff0, soff1]
        doff = [doff0, doff1]
        di0 = [di00, di01]
        di1 = [di10, di11]
        xb = [xb0, xb1]
        a0 = [a00, a01]
        a1 = [a10, a11]
        d0 = [d00, d01]
        d1 = [d10, d11]
        semg = [semg0, semg1]
        sems = [sems0, sems1]

        sc = lax.axis_index("c")
        tid = lax.axis_index("s")
        zv = jnp.zeros((16,), jnp.float32)

        @pl.loop(0, 46)
        def _(r):
            zba[r, pl.ds(0, 16)] = zv
            zba[r, pl.ds(16, 16)] = zv

        @pl.loop(0, 23)
        def _(i):
            zbs[pl.ds(i * 16, 16)] = zv

        r0 = tid * ROWS_PER_TILE
        zchunk = 368 if subh == 2 else 136
        nzc = ssw // zchunk
        @pl.loop(0, 68)
        def _(q):
            pltpu.async_copy(zba, acc_s.at[pl.ds(r0 + q * 46, 46), :], semg0)
        @pl.loop(0, nzc)
        def _(q):
            pltpu.async_copy(zbs.at[pl.ds(0, zchunk)],
                             ssum_s.at[pl.ds(tid * ssw + q * zchunk, zchunk)],
                             semg1)
        @pl.loop(0, 68)
        def _(q):
            pltpu.make_async_copy(
                zba, acc_s.at[pl.ds(r0 + q * 46, 46), :], semg0).wait()
        @pl.loop(0, nzc)
        def _(q):
            pltpu.make_async_copy(
                zbs.at[pl.ds(0, zchunk)],
                ssum_s.at[pl.ds(tid * ssw + q * zchunk, zchunk)], semg1).wait()
        plsc.subcore_barrier()

        src_off = sc * n_src
        dst_off = sc * NPAD
        erow0 = tid * rows_per_tile

        def refill(rowc):
            pltpu.sync_copy(srcw.at[pl.ds(rowc, 4 * C), :], sxc)
            pltpu.sync_copy(dstw.at[pl.ds(rowc, 4 * C), :], dxc)

        def derive(p, lo):
            for j in range(C):
                @pl.loop(0, 8)
                def _(cc):
                    s = sxc[lo + j, pl.ds(cc * 16, 16)]
                    soff[p][j, pl.ds(cc * 16, 16)] = s + src_off
                    d = dxc[lo + j, pl.ds(cc * 16, 16)]
                    doff[p][j, pl.ds(cc * 16, 16)] = d + dst_off
                    if subh == 2:
                        di0[p][j, pl.ds(cc * 16, 16)] = d * 2
                        di1[p][j, pl.ds(cc * 16, 16)] = d * 2 + 1
                    else:
                        di0[p][j, pl.ds(cc * 16, 16)] = d

        def gather_pairs(p):
            prs = []
            for j in range(C):
                prs.append((x2.at[soff[p].at[j]], xb[p].at[j]))
                prs.append((alsA.at[soff[p].at[j]], a0[p].at[j]))
                prs.append((aldA.at[doff[p].at[j]], d0[p].at[j]))
                if subh == 2:
                    prs.append((alsB.at[soff[p].at[j]], a1[p].at[j]))
                    prs.append((aldB.at[doff[p].at[j]], d1[p].at[j]))
            return prs

        def issue_gathers(p):
            for s_, t_ in gather_pairs(p):
                pltpu.async_copy(s_, t_, semg[p])

        def wait_gathers(p):
            for s_, t_ in gather_pairs(p):
                pltpu.make_async_copy(s_, t_, semg[p]).wait()

        def compute(p):
            for j in range(C):
                @pl.loop(0, 8)
                def _(cc):
                    al0 = a0[p][j, pl.ds(cc * 16, 16)] + d0[p][j, pl.ds(cc * 16, 16)]
                    a0[p][j, pl.ds(cc * 16, 16)] = jnp.exp(
                        jnp.where(al0 >= 0, al0, 0.2 * al0))
                    if subh == 2:
                        al1 = a1[p][j, pl.ds(cc * 16, 16)] + d1[p][j, pl.ds(cc * 16, 16)]
                        a1[p][j, pl.ds(cc * 16, 16)] = jnp.exp(
                            jnp.where(al1 >= 0, al1, 0.2 * al1))
            for j in range(C):
                @pl.loop(0, 8)
                def _(cc):
                    wv0 = a0[p][j, pl.ds(cc * 16, 16)]
                    wv1 = a1[p][j, pl.ds(cc * 16, 16)] if subh == 2 else wv0
                    for l in range(16):
                        e = cc * 16 + l
                        xb[p][j, e, pl.ds(0, 16)] = xb[p][j, e, pl.ds(0, 16)] * wv0[l]
                        xb[p][j, e, pl.ds(16, 16)] = xb[p][j, e, pl.ds(16, 16)] * wv1[l]

        def issue_scatters(p, lo):
            descs = []
            for j in range(C):
                descs.append(pltpu.async_copy(
                    xb[p].at[j], acc_s.at[dxc.at[lo + j]], sems[p], add=True))
                descs.append(pltpu.async_copy(
                    a0[p].at[j], ssum_s.at[di0[p].at[j]], sems[p], add=True))
                if subh == 2:
                    descs.append(pltpu.async_copy(
                        a1[p].at[j], ssum_s.at[di1[p].at[j]], sems[p], add=True))
            return descs

        refill(erow0)
        derive(0, 0)
        issue_gathers(0)
        derive(1, C)
        issue_gathers(1)

        @pl.loop(0, npairs)
        def _(g):
            lo = (g % 2) * (2 * C)
            wait_gathers(0)
            compute(0)
            sc0 = issue_scatters(0, lo)
            wait_gathers(1)
            compute(1)
            sc1 = issue_scatters(1, lo + C)
            for dsc in sc0:
                dsc.wait()
            for dsc in sc1:
                dsc.wait()
            gn = g + 1

            @pl.when(jnp.logical_and(gn % 2 == 0, gn < npairs))
            def _():
                refill(erow0 + 2 * C * gn)

            @pl.when(gn < npairs)
            def _():
                nlo = (gn % 2) * (2 * C)
                derive(0, nlo)
                issue_gathers(0)
                derive(1, nlo + C)
                issue_gathers(1)

        plsc.subcore_barrier()
        co1 = pltpu.async_copy(
            acc_s.at[pl.ds(r0, ROWS_PER_TILE), :],
            acc_o.at[pl.ds(r0, ROWS_PER_TILE), pl.ds(sc * 32, 32)], semg0)
        co2 = pltpu.async_copy(
            ssum_s.at[pl.ds(tid * ssw, ssw)],
            ssum_o.at[sc, pl.ds(tid * ssw, ssw)], semg1)
        co1.wait()
        co2.wait()

    mesh = plsc.VectorSubcoreMesh(core_axis_name="c", subcore_axis_name="s")
    f32, i32 = jnp.float32, jnp.int32
    idxb = pltpu.VMEM((C, 128), i32)
    fb = pltpu.VMEM((C, 128), f32)
    return pl.kernel(
        body,
        out_type=[jax.ShapeDtypeStruct((NPAD, 64), f32),
                  jax.ShapeDtypeStruct((2, NPAD * subh), f32)],
        mesh=mesh,
        compiler_params=pltpu.CompilerParams(use_tc_tiling_on_sc=False),
        scratch_types=[
            pltpu.VMEM((4 * C, 128), i32),
            pltpu.VMEM((4 * C, 128), i32),
            idxb, idxb,
            idxb, idxb,
            idxb, idxb,
            idxb, idxb,
            pltpu.VMEM((C, 128, 32), f32),
            pltpu.VMEM((C, 128, 32), f32),
            fb, fb, fb, fb,
            fb, fb, fb, fb,
            pltpu.VMEM((46, 32), f32),
            pltpu.VMEM((368,), f32),
            pltpu.VMEM_SHARED((NPAD, 32), f32),
            pltpu.VMEM_SHARED((NPAD * subh,), f32),
            pltpu.SemaphoreType.DMA,
            pltpu.SemaphoreType.DMA,
            pltpu.SemaphoreType.DMA,
            pltpu.SemaphoreType.DMA,
        ],
    )


def _pad_edges(src, dst, n_src, n_edges_pad):
    e = src.shape[0]
    npad = n_edges_pad - e
    if npad:
        ar = jnp.arange(npad, dtype=jnp.int32)
        src = jnp.concatenate([src, ar % n_src])
        dst = jnp.concatenate([dst, N_HOST + ar % 48])
    return src.reshape(-1, 128), dst.reshape(-1, 128)


def _split2(x):
    return jnp.concatenate([x[:, :32], x[:, 32:]], axis=0)


def _pad_dst_tab(col_a, col_b):
    z = jnp.zeros((NPAD - N_HOST,), jnp.float32)
    return jnp.concatenate([col_a, z, col_b, z])


def _rep_mat(subh):
    import numpy as np
    heads = 2 * subh
    d = 64 // heads
    m = np.zeros((heads, 64), np.float32)
    for h in range(heads):
        m[h, h * d:(h + 1) * d] = 1.0
    return jnp.asarray(m)


def _norm1_block(ah_ref, sh_ref, au_ref, su_ref, kw_ref, kb_ref, r_ref,
                 oh_ref, ou_ref, km_ref, cov_ref):
    i = pl.program_id(0)
    r = r_ref[...]
    ohh = jnp.maximum(ah_ref[...] / (jnp.dot(sh_ref[...], r) + 1e-16), 0.0)
    ouh = jnp.maximum(au_ref[...] / (jnp.dot(su_ref[...], r) + 1e-16), 0.0)
    oh_ref[...] = ohh
    ou_ref[...] = ouh
    th = jnp.tanh(jnp.dot(ohh, kw_ref[...],
                          preferred_element_type=jnp.float32) + kb_ref[...])
    tu = jnp.tanh(jnp.dot(ouh, kw_ref[...],
                          preferred_element_type=jnp.float32) + kb_ref[...])
    c = jnp.sum((su_ref[...][:, 0:1] > 0).astype(jnp.float32))

    @pl.when(i == 0)
    def _():
        km_ref[...] = jnp.zeros_like(km_ref)
        cov_ref[...] = jnp.zeros_like(cov_ref)

    km_ref[...] = km_ref[...] + jnp.stack([th.sum(axis=0), tu.sum(axis=0)])
    cov_ref[...] = cov_ref[...] + jnp.full((1, 1), 0.0, jnp.float32) + c


def _norm1(acc_hh, ss_hh, acc_uh, ss_uh, kw, kb, block=1000):
    nb = N_HOST // block
    r = _rep_mat(2)
    return pl.pallas_call(
        _norm1_block,
        grid=(nb,),
        in_specs=[
            pl.BlockSpec((block, 64), lambda i: (i, 0)),
            pl.BlockSpec((block, 4), lambda i: (i, 0)),
            pl.BlockSpec((block, 64), lambda i: (i, 0)),
            pl.BlockSpec((block, 4), lambda i: (i, 0)),
            pl.BlockSpec((64, 64), lambda i: (0, 0)),
            pl.BlockSpec((64,), lambda i: (0,)),
            pl.BlockSpec((4, 64), lambda i: (0, 0)),
        ],
        out_specs=[
            pl.BlockSpec((block, 64), lambda i: (i, 0)),
            pl.BlockSpec((block, 64), lambda i: (i, 0)),
            pl.BlockSpec((2, 64), lambda i: (0, 0)),
            pl.BlockSpec((1, 1), lambda i: (0, 0)),
        ],
        out_shape=[
            jax.ShapeDtypeStruct((N_HOST, 64), jnp.float32),
            jax.ShapeDtypeStruct((N_HOST, 64), jnp.float32),
            jax.ShapeDtypeStruct((2, 64), jnp.float32),
            jax.ShapeDtypeStruct((1, 1), jnp.float32),
        ],
    )(acc_hh, ss_hh, acc_uh, ss_uh, kw, kb, r)


def _combine2_block(oh_ref, ou_ref, at_ref, w_ref, b_ref, a_ref, y_ref, al_ref):
    h2 = jnp.maximum(at_ref[0, 0] * oh_ref[...] + at_ref[0, 1] * ou_ref[...], 0.0)
    y = jnp.dot(h2, w_ref[...], preferred_element_type=jnp.float32) + b_ref[...]
    y_ref[...] = y
    al_ref[...] = jnp.dot(y, a_ref[...], preferred_element_type=jnp.float32)


def _combine2(o_hh, o_uh, attn, w2, b2, A2, block=1000):
    nb = N_HOST // block
    return pl.pallas_call(
        _combine2_block,
        grid=(nb,),
        in_specs=[
            pl.BlockSpec((block, 64), lambda i: (i, 0)),
            pl.BlockSpec((block, 64), lambda i: (i, 0)),
            pl.BlockSpec((1, 2), lambda i: (0, 0)),
            pl.BlockSpec((64, 64), lambda i: (0, 0)),
            pl.BlockSpec((64,), lambda i: (0,)),
            pl.BlockSpec((64, 2), lambda i: (0, 0)),
        ],
        out_specs=[
            pl.BlockSpec((block, 64), lambda i: (i, 0)),
            pl.BlockSpec((block, 2), lambda i: (i, 0)),
        ],
        out_shape=[
            jax.ShapeDtypeStruct((N_HOST, 64), jnp.float32),
            jax.ShapeDtypeStruct((N_HOST, 2), jnp.float32),
        ],
    )(o_hh, o_uh, attn, w2, b2, A2)


def _final_block(o2_ref, kw_ref, kb_ref, m_ref):
    i = pl.program_id(0)
    o2 = o2_ref[...]
    t = jnp.tanh(jnp.dot(o2, kw_ref[...],
                         preferred_element_type=jnp.float32) + kb_ref[...])

    @pl.when(i == 0)
    def _():
        m_ref[...] = jnp.zeros_like(m_ref)

    m_ref[...] = m_ref[...] + jnp.stack([o2.sum(axis=0), t.sum(axis=0)])


def _final_sums(o2, kw, kb, block=1000):
    nb = N_HOST // block
    return pl.pallas_call(
        _final_block,
        grid=(nb,),
        in_specs=[
            pl.BlockSpec((block, 64), lambda i: (i, 0)),
            pl.BlockSpec((64, 64), lambda i: (0, 0)),
            pl.BlockSpec((64,), lambda i: (0,)),
        ],
        out_specs=pl.BlockSpec((2, 64), lambda i: (0, 0)),
        out_shape=jax.ShapeDtypeStruct((2, 64), jnp.float32),
    )(o2, kw, kb)


def kernel(x_host, x_user, edge_index_host_auth_host, edge_index_user_authenticates_to_host,
           proj_host_w1, proj_host_b1, proj_user_w1, proj_user_b1,
           att_src_hh1, att_dst_hh1, att_src_uh1, att_dst_uh1, k_lin_w1, k_lin_b1, q1,
           proj_host_w2, proj_host_b2, proj_user_w2, proj_user_b2,
           att_src_hh2, att_dst_hh2, att_src_uh2, att_dst_uh2, k_lin_w2, k_lin_b2, q2,
           proj_w, proj_b):
    ei_hh = edge_index_host_auth_host
    ei_uh = edge_index_user_authenticates_to_host
    n = N_HOST

    E_HH_PAD = 802816
    E_UH_PAD = 409600
    src_hh, dst_hh = _pad_edges(ei_hh[0], ei_hh[1], N_HOST, E_HH_PAD)
    src_uh, dst_uh = _pad_edges(ei_uh[0], ei_uh[1], N_USER, E_UH_PAD)

    agg_hh = _make_sc_agg(E_HH_PAD, N_HOST, 2)
    agg_uh = _make_sc_agg(E_UH_PAD, N_USER, 2)
    agg_hh2 = _make_sc_agg(E_HH_PAD, N_HOST, 1)

    A_host1 = jnp.concatenate(
        [_att_mat(att_src_hh1), _att_mat(att_dst_hh1), _att_mat(att_dst_uh1)], axis=1)
    yh, al_h = _project(x_host, proj_host_w1, proj_host_b1, A_host1)
    yu, als_uh = _project(x_user, proj_user_w1, proj_user_b1, _att_mat(att_src_uh1))

    acc_hh, ss_hh = agg_hh(src_hh, dst_hh, _split2(yh),
                           jnp.concatenate([al_h[:, 0], al_h[:, 2]]),
                           jnp.concatenate([al_h[:, 1], al_h[:, 3]]),
                           _pad_dst_tab(al_h[:, 4], al_h[:, 6]),
                           _pad_dst_tab(al_h[:, 5], al_h[:, 7]))
    acc_uh, ss_uh = agg_uh(src_uh, dst_uh, _split2(yu),
                           jnp.concatenate([als_uh[:, 0], als_uh[:, 2]]),
                           jnp.concatenate([als_uh[:, 1], als_uh[:, 3]]),
                           _pad_dst_tab(al_h[:, 8], al_h[:, 10]),
                           _pad_dst_tab(al_h[:, 9], al_h[:, 11]))
    s4_hh = jnp.concatenate([ss_hh[0].reshape(NPAD, 2), ss_hh[1].reshape(NPAD, 2)],
                            axis=1)
    s4_uh = jnp.concatenate([ss_uh[0].reshape(NPAD, 2), ss_uh[1].reshape(NPAD, 2)],
                            axis=1)

    o_hh, o_uh, km, cov = _norm1(acc_hh, s4_hh, acc_uh, s4_uh, k_lin_w1, k_lin_b1)
    kmat = km / n
    score = (q1[None, :] * kmat).sum(-1)
    attn = jax.nn.softmax(score).reshape(1, 2)

    A_host2 = jnp.concatenate([_att_mat(att_src_hh2), _att_mat(att_dst_hh2)], axis=1)
    yh2, al2 = _combine2(o_hh, o_uh, attn, proj_host_w2, proj_host_b2, A_host2)
    acc2, ss2 = agg_hh2(src_hh, dst_hh, _split2(yh2),
                        jnp.concatenate([al2[:, 0], al2[:, 0]]),
                        jnp.concatenate([al2[:, 0], al2[:, 0]]),
                        _pad_dst_tab(al2[:, 1], al2[:, 1]),
                        _pad_dst_tab(al2[:, 1], al2[:, 1]))
    o2 = jnp.maximum(acc2[:N_HOST] / (ss2[0, :N_HOST, None] + 1e-16), 0.0)

    m = _final_sums(o2, k_lin_w2, k_lin_b2)
    frac = cov[0, 0] / n
    v1 = jax.nn.relu(proj_user_b2)
    kmat_hh2 = m[1] / n
    kmat_uh2 = frac * jnp.tanh(v1 @ k_lin_w2 + k_lin_b2) + (1 - frac) * jnp.tanh(k_lin_b2)
    score2 = jnp.stack([(q2 * kmat_hh2).sum(), (q2 * kmat_uh2).sum()])
    attn2 = jax.nn.softmax(score2)
    emb = attn2[0] * (m[0] / n) + attn2[1] * (frac * v1)
    return emb @ proj_w + proj_b


# speedup vs baseline: 90.1625x; 1.0252x over previous
import functools
import jax
import jax.numpy as jnp
from jax import lax
from jax.experimental import pallas as pl
from jax.experimental.pallas import tpu as pltpu
from jax.experimental.pallas import tpu_sc as plsc

N_HOST = 50000
N_USER = 25000
NPAD = 50048
ROWS_PER_TILE = NPAD // 16
C = 2


def _proj_block(x_ref, w_ref, b_ref, a_ref, y_ref, al_ref):
    y = jnp.dot(x_ref[...], w_ref[...], preferred_element_type=jnp.float32)
    y = y + b_ref[...]
    y_ref[...] = y
    al_ref[...] = jnp.dot(y, a_ref[...], preferred_element_type=jnp.float32)


def _project(x, w, b, A, block=1000):
    n, din = x.shape
    dout = w.shape[1]
    k = A.shape[1]
    return pl.pallas_call(
        _proj_block,
        grid=(n // block,),
        in_specs=[
            pl.BlockSpec((block, din), lambda i: (i, 0)),
            pl.BlockSpec((din, dout), lambda i: (0, 0)),
            pl.BlockSpec((dout,), lambda i: (0,)),
            pl.BlockSpec((dout, k), lambda i: (0, 0)),
        ],
        out_specs=[
            pl.BlockSpec((block, dout), lambda i: (i, 0)),
            pl.BlockSpec((block, k), lambda i: (i, 0)),
        ],
        out_shape=[
            jax.ShapeDtypeStruct((n, dout), jnp.float32),
            jax.ShapeDtypeStruct((n, k), jnp.float32),
        ],
    )(x, w, b, A)


def _att_mat(a):
    heads, d = a.shape
    hsel = jnp.repeat(jnp.eye(heads, dtype=a.dtype), d, axis=0)
    return hsel * a.reshape(-1)[:, None]


def _make_sc_agg(n_edges_pad, n_src, subh):
    rows = n_edges_pad // 128
    rows_per_tile = rows // 16
    npairs = rows_per_tile // (2 * C)
    ssw = ROWS_PER_TILE * subh

    def body(srcw, dstw, x2, alsA, alsB, aldA, aldB, acc_o, ssum_o,
             sxc, dxc,
             soff0, soff1, doff0, doff1, di00, di01, di10, di11,
             xb0, xb1, a00, a01, a10, a11, d00, d01, d10, d11,
             zba, zbs, acc_s, ssum_s, semg0, semg1, sems0, sems1):
        soff = [soff0, soff1]
        doff = [doff0, doff1]
        di0 = [di00, di01]
        di1 = [di10, di11]
        xb = [xb0, xb1]
        a0 = [a00, a01]
        a1 = [a10, a11]
        d0 = [d00, d01]
        d1 = [d10, d11]
        semg = [semg0, semg1]
        sems = [sems0, sems1]

        sc = lax.axis_index("c")
        tid = lax.axis_index("s")
        zv = jnp.zeros((16,), jnp.float32)

        @pl.loop(0, 46)
        def _(r):
            zba[r, pl.ds(0, 16)] = zv
            zba[r, pl.ds(16, 16)] = zv

        @pl.loop(0, 23)
        def _(i):
            zbs[pl.ds(i * 16, 16)] = zv

        r0 = tid * ROWS_PER_TILE
        zchunk = 368 if subh == 2 else 136
        nzc = ssw // zchunk
        @pl.loop(0, 68)
        def _(q):
            pltpu.async_copy(zba, acc_s.at[pl.ds(r0 + q * 46, 46), :], semg0)
        @pl.loop(0, nzc)
        def _(q):
            pltpu.async_copy(zbs.at[pl.ds(0, zchunk)],
                             ssum_s.at[pl.ds(tid * ssw + q * zchunk, zchunk)],
                             semg1)
        @pl.loop(0, 68)
        def _(q):
            pltpu.make_async_copy(
                zba, acc_s.at[pl.ds(r0 + q * 46, 46), :], semg0).wait()
        @pl.loop(0, nzc)
        def _(q):
            pltpu.make_async_copy(
                zbs.at[pl.ds(0, zchunk)],
                ssum_s.at[pl.ds(tid * ssw + q * zchunk, zchunk)], semg1).wait()
        plsc.subcore_barrier()

        src_off = sc * n_src
        dst_off = sc * NPAD
        erow0 = tid * rows_per_tile

        def refill(rowc):
            descs = []
            for r_ in range(4 * C):
                descs.append(pltpu.async_copy(
                    srcw.at[pl.ds((rowc + r_) * 128, 128)], sxc.at[r_], semg0))
                descs.append(pltpu.async_copy(
                    dstw.at[pl.ds((rowc + r_) * 128, 128)], dxc.at[r_], semg1))
            for d_ in descs:
                d_.wait()

        def derive(p, lo):
            for j in range(C):
                @pl.loop(0, 8)
                def _(cc):
                    s = sxc[lo + j, pl.ds(cc * 16, 16)]
                    soff[p][j, pl.ds(cc * 16, 16)] = s + src_off
                    d = dxc[lo + j, pl.ds(cc * 16, 16)]
                    doff[p][j, pl.ds(cc * 16, 16)] = d + dst_off
                    if subh == 2:
                        di0[p][j, pl.ds(cc * 16, 16)] = d * 2
                        di1[p][j, pl.ds(cc * 16, 16)] = d * 2 + 1
                    else:
                        di0[p][j, pl.ds(cc * 16, 16)] = d

        def gather_pairs(p):
            prs = []
            for j in range(C):
                prs.append((x2.at[soff[p].at[j]], xb[p].at[j]))
                prs.append((alsA.at[soff[p].at[j]], a0[p].at[j]))
                prs.append((aldA.at[doff[p].at[j]], d0[p].at[j]))
                if subh == 2:
                    prs.append((alsB.at[soff[p].at[j]], a1[p].at[j]))
                    prs.append((aldB.at[doff[p].at[j]], d1[p].at[j]))
            return prs

        def issue_gathers(p):
            for s_, t_ in gather_pairs(p):
                pltpu.async_copy(s_, t_, semg[p])

        def wait_gathers(p):
            for s_, t_ in gather_pairs(p):
                pltpu.make_async_copy(s_, t_, semg[p]).wait()

        def compute(p):
            for j in range(C):
                @pl.loop(0, 8)
                def _(cc):
                    al0 = a0[p][j, pl.ds(cc * 16, 16)] + d0[p][j, pl.ds(cc * 16, 16)]
                    a0[p][j, pl.ds(cc * 16, 16)] = jnp.exp(
                        jnp.where(al0 >= 0, al0, 0.2 * al0))
                    if subh == 2:
                        al1 = a1[p][j, pl.ds(cc * 16, 16)] + d1[p][j, pl.ds(cc * 16, 16)]
                        a1[p][j, pl.ds(cc * 16, 16)] = jnp.exp(
                            jnp.where(al1 >= 0, al1, 0.2 * al1))
            for j in range(C):
                @pl.loop(0, 8)
                def _(cc):
                    wv0 = a0[p][j, pl.ds(cc * 16, 16)]
                    wv1 = a1[p][j, pl.ds(cc * 16, 16)] if subh == 2 else wv0
                    for l in range(16):
                        e = cc * 16 + l
                        xb[p][j, e, pl.ds(0, 16)] = xb[p][j, e, pl.ds(0, 16)] * wv0[l]
                        xb[p][j, e, pl.ds(16, 16)] = xb[p][j, e, pl.ds(16, 16)] * wv1[l]

        def issue_scatters(p, lo):
            descs = []
            for j in range(C):
                descs.append(pltpu.async_copy(
                    xb[p].at[j], acc_s.at[dxc.at[lo + j]], sems[p], add=True))
                descs.append(pltpu.async_copy(
                    a0[p].at[j], ssum_s.at[di0[p].at[j]], sems[p], add=True))
                if subh == 2:
                    descs.append(pltpu.async_copy(
                        a1[p].at[j], ssum_s.at[di1[p].at[j]], sems[p], add=True))
            return descs

        refill(erow0)
        derive(0, 0)
        issue_gathers(0)
        derive(1, C)
        issue_gathers(1)

        @pl.loop(0, npairs)
        def _(g):
            lo = (g % 2) * (2 * C)
            wait_gathers(0)
            compute(0)
            sc0 = issue_scatters(0, lo)
            wait_gathers(1)
            compute(1)
            sc1 = issue_scatters(1, lo + C)
            for dsc in sc0:
                dsc.wait()
            for dsc in sc1:
                dsc.wait()
            gn = g + 1

            @pl.when(jnp.logical_and(gn % 2 == 0, gn < npairs))
            def _():
                refill(erow0 + 2 * C * gn)

            @pl.when(gn < npairs)
            def _():
                nlo = (gn % 2) * (2 * C)
                derive(0, nlo)
                issue_gathers(0)
                derive(1, nlo + C)
                issue_gathers(1)

        plsc.subcore_barrier()
        co1 = pltpu.async_copy(
            acc_s.at[pl.ds(r0, ROWS_PER_TILE), :],
            acc_o.at[pl.ds(r0, ROWS_PER_TILE), pl.ds(sc * 32, 32)], semg0)
        co2 = pltpu.async_copy(
            ssum_s.at[pl.ds(tid * ssw, ssw)],
            ssum_o.at[sc, pl.ds(tid * ssw, ssw)], semg1)
        co1.wait()
        co2.wait()

    mesh = plsc.VectorSubcoreMesh(core_axis_name="c", subcore_axis_name="s")
    f32, i32 = jnp.float32, jnp.int32
    idxb = pltpu.VMEM((C, 128), i32)
    fb = pltpu.VMEM((C, 128), f32)
    return pl.kernel(
        body,
        out_type=[jax.ShapeDtypeStruct((NPAD, 64), f32),
                  jax.ShapeDtypeStruct((2, NPAD * subh), f32)],
        mesh=mesh,
        compiler_params=pltpu.CompilerParams(use_tc_tiling_on_sc=False),
        scratch_types=[
            pltpu.VMEM((4 * C, 128), i32),
            pltpu.VMEM((4 * C, 128), i32),
            idxb, idxb,
            idxb, idxb,
            idxb, idxb,
            idxb, idxb,
            pltpu.VMEM((C, 128, 32), f32),
            pltpu.VMEM((C, 128, 32), f32),
            fb, fb, fb, fb,
            fb, fb, fb, fb,
            pltpu.VMEM((46, 32), f32),
            pltpu.VMEM((368,), f32),
            pltpu.VMEM_SHARED((NPAD, 32), f32),
            pltpu.VMEM_SHARED((NPAD * subh,), f32),
            pltpu.SemaphoreType.DMA,
            pltpu.SemaphoreType.DMA,
            pltpu.SemaphoreType.DMA,
            pltpu.SemaphoreType.DMA,
        ],
    )


def _pad_edges(src, dst, n_src, n_edges_pad):
    e = src.shape[0]
    npad = n_edges_pad - e
    if npad:
        ar = jnp.arange(npad, dtype=jnp.int32)
        src = jnp.concatenate([src, ar % n_src])
        dst = jnp.concatenate([dst, N_HOST + ar % 48])
    return src, dst


def _split2(x):
    return jnp.concatenate([x[:, :32], x[:, 32:]], axis=0)


def _pad_dst_tab(col_a, col_b):
    z = jnp.zeros((NPAD - N_HOST,), jnp.float32)
    return jnp.concatenate([col_a, z, col_b, z])


def _rep_mat(subh):
    import numpy as np
    heads = 2 * subh
    d = 64 // heads
    m = np.zeros((heads, 64), np.float32)
    for h in range(heads):
        m[h, h * d:(h + 1) * d] = 1.0
    return jnp.asarray(m)


def _norm1_block(ah_ref, sh_ref, au_ref, su_ref, kw_ref, kb_ref, r_ref,
                 oh_ref, ou_ref, km_ref, cov_ref):
    i = pl.program_id(0)
    r = r_ref[...]
    ohh = jnp.maximum(ah_ref[...] / (jnp.dot(sh_ref[...], r) + 1e-16), 0.0)
    ouh = jnp.maximum(au_ref[...] / (jnp.dot(su_ref[...], r) + 1e-16), 0.0)
    oh_ref[...] = ohh
    ou_ref[...] = ouh
    th = jnp.tanh(jnp.dot(ohh, kw_ref[...],
                          preferred_element_type=jnp.float32) + kb_ref[...])
    tu = jnp.tanh(jnp.dot(ouh, kw_ref[...],
                          preferred_element_type=jnp.float32) + kb_ref[...])
    c = jnp.sum((su_ref[...][:, 0:1] > 0).astype(jnp.float32))

    @pl.when(i == 0)
    def _():
        km_ref[...] = jnp.zeros_like(km_ref)
        cov_ref[...] = jnp.zeros_like(cov_ref)

    km_ref[...] = km_ref[...] + jnp.stack([th.sum(axis=0), tu.sum(axis=0)])
    cov_ref[...] = cov_ref[...] + jnp.full((1, 1), 0.0, jnp.float32) + c


def _norm1(acc_hh, ss_hh, acc_uh, ss_uh, kw, kb, block=1000):
    nb = N_HOST // block
    r = _rep_mat(2)
    return pl.pallas_call(
        _norm1_block,
        grid=(nb,),
        in_specs=[
            pl.BlockSpec((block, 64), lambda i: (i, 0)),
            pl.BlockSpec((block, 4), lambda i: (i, 0)),
            pl.BlockSpec((block, 64), lambda i: (i, 0)),
            pl.BlockSpec((block, 4), lambda i: (i, 0)),
            pl.BlockSpec((64, 64), lambda i: (0, 0)),
            pl.BlockSpec((64,), lambda i: (0,)),
            pl.BlockSpec((4, 64), lambda i: (0, 0)),
        ],
        out_specs=[
            pl.BlockSpec((block, 64), lambda i: (i, 0)),
            pl.BlockSpec((block, 64), lambda i: (i, 0)),
            pl.BlockSpec((2, 64), lambda i: (0, 0)),
            pl.BlockSpec((1, 1), lambda i: (0, 0)),
        ],
        out_shape=[
            jax.ShapeDtypeStruct((N_HOST, 64), jnp.float32),
            jax.ShapeDtypeStruct((N_HOST, 64), jnp.float32),
            jax.ShapeDtypeStruct((2, 64), jnp.float32),
            jax.ShapeDtypeStruct((1, 1), jnp.float32),
        ],
    )(acc_hh, ss_hh, acc_uh, ss_uh, kw, kb, r)


def _combine2_block(oh_ref, ou_ref, at_ref, w_ref, b_ref, a_ref, y_ref, al_ref):
    h2 = jnp.maximum(at_ref[0, 0] * oh_ref[...] + at_ref[0, 1] * ou_ref[...], 0.0)
    y = jnp.dot(h2, w_ref[...], preferred_element_type=jnp.float32) + b_ref[...]
    y_ref[...] = y
    al_ref[...] = jnp.dot(y, a_ref[...], preferred_element_type=jnp.float32)


def _combine2(o_hh, o_uh, attn, w2, b2, A2, block=1000):
    nb = N_HOST // block
    return pl.pallas_call(
        _combine2_block,
        grid=(nb,),
        in_specs=[
            pl.BlockSpec((block, 64), lambda i: (i, 0)),
            pl.BlockSpec((block, 64), lambda i: (i, 0)),
            pl.BlockSpec((1, 2), lambda i: (0, 0)),
            pl.BlockSpec((64, 64), lambda i: (0, 0)),
            pl.BlockSpec((64,), lambda i: (0,)),
            pl.BlockSpec((64, 2), lambda i: (0, 0)),
        ],
        out_specs=[
            pl.BlockSpec((block, 64), lambda i: (i, 0)),
            pl.BlockSpec((block, 2), lambda i: (i, 0)),
        ],
        out_shape=[
            jax.ShapeDtypeStruct((N_HOST, 64), jnp.float32),
            jax.ShapeDtypeStruct((N_HOST, 2), jnp.float32),
        ],
    )(o_hh, o_uh, attn, w2, b2, A2)


def _final_block(o2_ref, kw_ref, kb_ref, m_ref):
    i = pl.program_id(0)
    o2 = o2_ref[...]
    t = jnp.tanh(jnp.dot(o2, kw_ref[...],
                         preferred_element_type=jnp.float32) + kb_ref[...])

    @pl.when(i == 0)
    def _():
        m_ref[...] = jnp.zeros_like(m_ref)

    m_ref[...] = m_ref[...] + jnp.stack([o2.sum(axis=0), t.sum(axis=0)])


def _final_sums(o2, kw, kb, block=1000):
    nb = N_HOST // block
    return pl.pallas_call(
        _final_block,
        grid=(nb,),
        in_specs=[
            pl.BlockSpec((block, 64), lambda i: (i, 0)),
            pl.BlockSpec((64, 64), lambda i: (0, 0)),
            pl.BlockSpec((64,), lambda i: (0,)),
        ],
        out_specs=pl.BlockSpec((2, 64), lambda i: (0, 0)),
        out_shape=jax.ShapeDtypeStruct((2, 64), jnp.float32),
    )(o2, kw, kb)


def kernel(x_host, x_user, edge_index_host_auth_host, edge_index_user_authenticates_to_host,
           proj_host_w1, proj_host_b1, proj_user_w1, proj_user_b1,
           att_src_hh1, att_dst_hh1, att_src_uh1, att_dst_uh1, k_lin_w1, k_lin_b1, q1,
           proj_host_w2, proj_host_b2, proj_user_w2, proj_user_b2,
           att_src_hh2, att_dst_hh2, att_src_uh2, att_dst_uh2, k_lin_w2, k_lin_b2, q2,
           proj_w, proj_b):
    ei_hh = edge_index_host_auth_host
    ei_uh = edge_index_user_authenticates_to_host
    n = N_HOST

    E_HH_PAD = 802816
    E_UH_PAD = 409600
    src_hh, dst_hh = _pad_edges(ei_hh[0], ei_hh[1], N_HOST, E_HH_PAD)
    src_uh, dst_uh = _pad_edges(ei_uh[0], ei_uh[1], N_USER, E_UH_PAD)

    agg_hh = _make_sc_agg(E_HH_PAD, N_HOST, 2)
    agg_uh = _make_sc_agg(E_UH_PAD, N_USER, 2)
    agg_hh2 = _make_sc_agg(E_HH_PAD, N_HOST, 1)

    A_host1 = jnp.concatenate(
        [_att_mat(att_src_hh1), _att_mat(att_dst_hh1), _att_mat(att_dst_uh1)], axis=1)
    yh, al_h = _project(x_host, proj_host_w1, proj_host_b1, A_host1)
    yu, als_uh = _project(x_user, proj_user_w1, proj_user_b1, _att_mat(att_src_uh1))

    acc_hh, ss_hh = agg_hh(src_hh, dst_hh, _split2(yh),
                           jnp.concatenate([al_h[:, 0], al_h[:, 2]]),
                           jnp.concatenate([al_h[:, 1], al_h[:, 3]]),
                           _pad_dst_tab(al_h[:, 4], al_h[:, 6]),
                           _pad_dst_tab(al_h[:, 5], al_h[:, 7]))
    acc_uh, ss_uh = agg_uh(src_uh, dst_uh, _split2(yu),
                           jnp.concatenate([als_uh[:, 0], als_uh[:, 2]]),
                           jnp.concatenate([als_uh[:, 1], als_uh[:, 3]]),
                           _pad_dst_tab(al_h[:, 8], al_h[:, 10]),
                           _pad_dst_tab(al_h[:, 9], al_h[:, 11]))
    s4_hh = jnp.concatenate([ss_hh[0].reshape(NPAD, 2), ss_hh[1].reshape(NPAD, 2)],
                            axis=1)
    s4_uh = jnp.concatenate([ss_uh[0].reshape(NPAD, 2), ss_uh[1].reshape(NPAD, 2)],
                            axis=1)

    o_hh, o_uh, km, cov = _norm1(acc_hh, s4_hh, acc_uh, s4_uh, k_lin_w1, k_lin_b1)
    kmat = km / n
    score = (q1[None, :] * kmat).sum(-1)
    attn = jax.nn.softmax(score).reshape(1, 2)

    A_host2 = jnp.concatenate([_att_mat(att_src_hh2), _att_mat(att_dst_hh2)], axis=1)
    yh2, al2 = _combine2(o_hh, o_uh, attn, proj_host_w2, proj_host_b2, A_host2)
    acc2, ss2 = agg_hh2(src_hh, dst_hh, _split2(yh2),
                        jnp.concatenate([al2[:, 0], al2[:, 0]]),
                        jnp.concatenate([al2[:, 0], al2[:, 0]]),
                        _pad_dst_tab(al2[:, 1], al2[:, 1]),
                        _pad_dst_tab(al2[:, 1], al2[:, 1]))
    o2 = jnp.maximum(acc2[:N_HOST] / (ss2[0, :N_HOST, None] + 1e-16), 0.0)

    m = _final_sums(o2, k_lin_w2, k_lin_b2)
    frac = cov[0, 0] / n
    v1 = jax.nn.relu(proj_user_b2)
    kmat_hh2 = m[1] / n
    kmat_uh2 = frac * jnp.tanh(v1 @ k_lin_w2 + k_lin_b2) + (1 - frac) * jnp.tanh(k_lin_b2)
    score2 = jnp.stack([(q2 * kmat_hh2).sum(), (q2 * kmat_uh2).sum()])
    attn2 = jax.nn.softmax(score2)
    emb = attn2[0] * (m[0] / n) + attn2[1] * (frac * v1)
    return emb @ proj_w + proj_b


# speedup vs baseline: 91.3497x; 1.0132x over previous
import functools
import jax
import jax.numpy as jnp
from jax import lax
from jax.experimental import pallas as pl
from jax.experimental.pallas import tpu as pltpu
from jax.experimental.pallas import tpu_sc as plsc

N_HOST = 50000
N_USER = 25000
NPAD = 50048
ROWS_PER_TILE = NPAD // 16
C = 2


def _proj_block(x_ref, w_ref, b_ref, a_ref, ys_ref, al_ref):
    i = pl.program_id(0)
    h = i % 2
    y = jnp.dot(x_ref[...], w_ref[...], preferred_element_type=jnp.float32)
    y = y + b_ref[...]
    ys_ref[...] = jnp.where(h == 0, y[:, :32], y[:, 32:])
    al_ref[...] = jnp.dot(y, a_ref[...], preferred_element_type=jnp.float32)


def _project(x, w, b, A, block=1000):
    n, din = x.shape
    dout = w.shape[1]
    k = A.shape[1]
    nb = n // block
    return pl.pallas_call(
        _proj_block,
        grid=(2 * nb,),
        in_specs=[
            pl.BlockSpec((block, din), lambda i: (i // 2, 0)),
            pl.BlockSpec((din, dout), lambda i: (0, 0)),
            pl.BlockSpec((dout,), lambda i: (0,)),
            pl.BlockSpec((dout, k), lambda i: (0, 0)),
        ],
        out_specs=[
            pl.BlockSpec((block, 32), lambda i: ((i % 2) * (n // block) + i // 2, 0)),
            pl.BlockSpec((block, k), lambda i: (i // 2, 0)),
        ],
        out_shape=[
            jax.ShapeDtypeStruct((2 * n, 32), jnp.float32),
            jax.ShapeDtypeStruct((n, k), jnp.float32),
        ],
    )(x, w, b, A)


def _att_mat(a):
    heads, d = a.shape
    hsel = jnp.repeat(jnp.eye(heads, dtype=a.dtype), d, axis=0)
    return hsel * a.reshape(-1)[:, None]


def _make_sc_agg(n_edges_pad, n_src, subh):
    rows = n_edges_pad // 128
    rows_per_tile = rows // 16
    npairs = rows_per_tile // (2 * C)
    ssw = ROWS_PER_TILE * subh

    def body(srcw, dstw, x2, alsA, alsB, aldA, aldB, acc_o, ssum_o,
             sxc, dxc,
             soff0, soff1, doff0, doff1, di00, di01, di10, di11,
             xb0, xb1, a00, a01, a10, a11, d00, d01, d10, d11,
             zba, zbs, acc_s, ssum_s, semg0, semg1, sems0, sems1):
        soff = [soff0, soff1]
        doff = [doff0, doff1]
        di0 = [di00, di01]
        di1 = [di10, di11]
        xb = [xb0, xb1]
        a0 = [a00, a01]
        a1 = [a10, a11]
        d0 = [d00, d01]
        d1 = [d10, d11]
        semg = [semg0, semg1]
        sems = [sems0, sems1]

        sc = lax.axis_index("c")
        tid = lax.axis_index("s")
        zv = jnp.zeros((16,), jnp.float32)

        @pl.loop(0, 46)
        def _(r):
            zba[r, pl.ds(0, 16)] = zv
            zba[r, pl.ds(16, 16)] = zv

        @pl.loop(0, 23)
        def _(i):
            zbs[pl.ds(i * 16, 16)] = zv

        r0 = tid * ROWS_PER_TILE
        zchunk = 368 if subh == 2 else 136
        nzc = ssw // zchunk
        @pl.loop(0, 68)
        def _(q):
            pltpu.async_copy(zba, acc_s.at[pl.ds(r0 + q * 46, 46), :], semg0)
        @pl.loop(0, nzc)
        def _(q):
            pltpu.async_copy(zbs.at[pl.ds(0, zchunk)],
                             ssum_s.at[pl.ds(tid * ssw + q * zchunk, zchunk)],
                             semg1)
        @pl.loop(0, 68)
        def _(q):
            pltpu.make_async_copy(
                zba, acc_s.at[pl.ds(r0 + q * 46, 46), :], semg0).wait()
        @pl.loop(0, nzc)
        def _(q):
            pltpu.make_async_copy(
                zbs.at[pl.ds(0, zchunk)],
                ssum_s.at[pl.ds(tid * ssw + q * zchunk, zchunk)], semg1).wait()
        plsc.subcore_barrier()

        src_off = sc * n_src
        dst_off = sc * NPAD
        erow0 = tid * rows_per_tile

        def refill(rowc):
            descs = []
            for r_ in range(4 * C):
                descs.append(pltpu.async_copy(
                    srcw.at[pl.ds((rowc + r_) * 128, 128)], sxc.at[r_], semg0))
                descs.append(pltpu.async_copy(
                    dstw.at[pl.ds((rowc + r_) * 128, 128)], dxc.at[r_], semg1))
            for d_ in descs:
                d_.wait()

        def derive(p, lo):
            for j in range(C):
                @pl.loop(0, 8)
                def _(cc):
                    s = sxc[lo + j, pl.ds(cc * 16, 16)]
                    soff[p][j, pl.ds(cc * 16, 16)] = s + src_off
                    d = dxc[lo + j, pl.ds(cc * 16, 16)]
                    doff[p][j, pl.ds(cc * 16, 16)] = d + dst_off
                    if subh == 2:
                        di0[p][j, pl.ds(cc * 16, 16)] = d * 2
                        di1[p][j, pl.ds(cc * 16, 16)] = d * 2 + 1
                    else:
                        di0[p][j, pl.ds(cc * 16, 16)] = d

        def gather_pairs(p):
            prs = []
            for j in range(C):
                prs.append((x2.at[soff[p].at[j]], xb[p].at[j]))
                prs.append((alsA.at[soff[p].at[j]], a0[p].at[j]))
                prs.append((aldA.at[doff[p].at[j]], d0[p].at[j]))
                if subh == 2:
                    prs.append((alsB.at[soff[p].at[j]], a1[p].at[j]))
                    prs.append((aldB.at[doff[p].at[j]], d1[p].at[j]))
            return prs

        def issue_gathers(p):
            for s_, t_ in gather_pairs(p):
                pltpu.async_copy(s_, t_, semg[p])

        def wait_gathers(p):
            for s_, t_ in gather_pairs(p):
                pltpu.make_async_copy(s_, t_, semg[p]).wait()

        def compute(p):
            for j in range(C):
                @pl.loop(0, 8)
                def _(cc):
                    al0 = a0[p][j, pl.ds(cc * 16, 16)] + d0[p][j, pl.ds(cc * 16, 16)]
                    a0[p][j, pl.ds(cc * 16, 16)] = jnp.exp(
                        jnp.where(al0 >= 0, al0, 0.2 * al0))
                    if subh == 2:
                        al1 = a1[p][j, pl.ds(cc * 16, 16)] + d1[p][j, pl.ds(cc * 16, 16)]
                        a1[p][j, pl.ds(cc * 16, 16)] = jnp.exp(
                            jnp.where(al1 >= 0, al1, 0.2 * al1))
            for j in range(C):
                @pl.loop(0, 8)
                def _(cc):
                    wv0 = a0[p][j, pl.ds(cc * 16, 16)]
                    wv1 = a1[p][j, pl.ds(cc * 16, 16)] if subh == 2 else wv0
                    for l in range(16):
                        e = cc * 16 + l
                        xb[p][j, e, pl.ds(0, 16)] = xb[p][j, e, pl.ds(0, 16)] * wv0[l]
                        xb[p][j, e, pl.ds(16, 16)] = xb[p][j, e, pl.ds(16, 16)] * wv1[l]

        def issue_scatters(p, lo):
            descs = []
            for j in range(C):
                descs.append(pltpu.async_copy(
                    xb[p].at[j], acc_s.at[dxc.at[lo + j]], sems[p], add=True))
                descs.append(pltpu.async_copy(
                    a0[p].at[j], ssum_s.at[di0[p].at[j]], sems[p], add=True))
                if subh == 2:
                    descs.append(pltpu.async_copy(
                        a1[p].at[j], ssum_s.at[di1[p].at[j]], sems[p], add=True))
            return descs

        refill(erow0)
        derive(0, 0)
        issue_gathers(0)
        derive(1, C)
        issue_gathers(1)

        @pl.loop(0, npairs)
        def _(g):
            lo = (g % 2) * (2 * C)
            wait_gathers(0)
            compute(0)
            sc0 = issue_scatters(0, lo)
            wait_gathers(1)
            compute(1)
            sc1 = issue_scatters(1, lo + C)
            for dsc in sc0:
                dsc.wait()
            for dsc in sc1:
                dsc.wait()
            gn = g + 1

            @pl.when(jnp.logical_and(gn % 2 == 0, gn < npairs))
            def _():
                refill(erow0 + 2 * C * gn)

            @pl.when(gn < npairs)
            def _():
                nlo = (gn % 2) * (2 * C)
                derive(0, nlo)
                issue_gathers(0)
                derive(1, nlo + C)
                issue_gathers(1)

        plsc.subcore_barrier()
        co1 = pltpu.async_copy(
            acc_s.at[pl.ds(r0, ROWS_PER_TILE), :],
            acc_o.at[pl.ds(r0, ROWS_PER_TILE), pl.ds(sc * 32, 32)], semg0)
        co2 = pltpu.async_copy(
            ssum_s.at[pl.ds(tid * ssw, ssw)],
            ssum_o.at[sc, pl.ds(tid * ssw, ssw)], semg1)
        co1.wait()
        co2.wait()

    mesh = plsc.VectorSubcoreMesh(core_axis_name="c", subcore_axis_name="s")
    f32, i32 = jnp.float32, jnp.int32
    idxb = pltpu.VMEM((C, 128), i32)
    fb = pltpu.VMEM((C, 128), f32)
    return pl.kernel(
        body,
        out_type=[jax.ShapeDtypeStruct((NPAD, 64), f32),
                  jax.ShapeDtypeStruct((2, NPAD * subh), f32)],
        mesh=mesh,
        compiler_params=pltpu.CompilerParams(use_tc_tiling_on_sc=False),
        scratch_types=[
            pltpu.VMEM((4 * C, 128), i32),
            pltpu.VMEM((4 * C, 128), i32),
            idxb, idxb,
            idxb, idxb,
            idxb, idxb,
            idxb, idxb,
            pltpu.VMEM((C, 128, 32), f32),
            pltpu.VMEM((C, 128, 32), f32),
            fb, fb, fb, fb,
            fb, fb, fb, fb,
            pltpu.VMEM((46, 32), f32),
            pltpu.VMEM((368,), f32),
            pltpu.VMEM_SHARED((NPAD, 32), f32),
            pltpu.VMEM_SHARED((NPAD * subh,), f32),
            pltpu.SemaphoreType.DMA,
            pltpu.SemaphoreType.DMA,
            pltpu.SemaphoreType.DMA,
            pltpu.SemaphoreType.DMA,
        ],
    )


def _pad_edges(src, dst, n_src, n_edges_pad):
    e = src.shape[0]
    npad = n_edges_pad - e
    if npad:
        ar = jnp.arange(npad, dtype=jnp.int32)
        src = jnp.concatenate([src, ar % n_src])
        dst = jnp.concatenate([dst, N_HOST + ar % 48])
    return src, dst


def _split2(x):
    return jnp.concatenate([x[:, :32], x[:, 32:]], axis=0)


def _pad_dst_tab(col_a, col_b):
    z = jnp.zeros((NPAD - N_HOST,), jnp.float32)
    return jnp.concatenate([col_a, z, col_b, z])


def _rep_mat(subh):
    import numpy as np
    heads = 2 * subh
    d = 64 // heads
    m = np.zeros((heads, 64), np.float32)
    for h in range(heads):
        m[h, h * d:(h + 1) * d] = 1.0
    return jnp.asarray(m)


def _norm1_block(ah_ref, sh_ref, au_ref, su_ref, kw_ref, kb_ref, r_ref,
                 oh_ref, ou_ref, km_ref, cov_ref):
    i = pl.program_id(0)
    r = r_ref[...]
    ohh = jnp.maximum(ah_ref[...] / (jnp.dot(sh_ref[...], r) + 1e-16), 0.0)
    ouh = jnp.maximum(au_ref[...] / (jnp.dot(su_ref[...], r) + 1e-16), 0.0)
    oh_ref[...] = ohh
    ou_ref[...] = ouh
    th = jnp.tanh(jnp.dot(ohh, kw_ref[...],
                          preferred_element_type=jnp.float32) + kb_ref[...])
    tu = jnp.tanh(jnp.dot(ouh, kw_ref[...],
                          preferred_element_type=jnp.float32) + kb_ref[...])
    c = jnp.sum((su_ref[...][:, 0:1] > 0).astype(jnp.float32))

    @pl.when(i == 0)
    def _():
        km_ref[...] = jnp.zeros_like(km_ref)
        cov_ref[...] = jnp.zeros_like(cov_ref)

    km_ref[...] = km_ref[...] + jnp.stack([th.sum(axis=0), tu.sum(axis=0)])
    cov_ref[...] = cov_ref[...] + jnp.full((1, 1), 0.0, jnp.float32) + c


def _norm1(acc_hh, ss_hh, acc_uh, ss_uh, kw, kb, block=1000):
    nb = N_HOST // block
    r = _rep_mat(2)
    return pl.pallas_call(
        _norm1_block,
        grid=(nb,),
        in_specs=[
            pl.BlockSpec((block, 64), lambda i: (i, 0)),
            pl.BlockSpec((block, 4), lambda i: (i, 0)),
            pl.BlockSpec((block, 64), lambda i: (i, 0)),
            pl.BlockSpec((block, 4), lambda i: (i, 0)),
            pl.BlockSpec((64, 64), lambda i: (0, 0)),
            pl.BlockSpec((64,), lambda i: (0,)),
            pl.BlockSpec((4, 64), lambda i: (0, 0)),
        ],
        out_specs=[
            pl.BlockSpec((block, 64), lambda i: (i, 0)),
            pl.BlockSpec((block, 64), lambda i: (i, 0)),
            pl.BlockSpec((2, 64), lambda i: (0, 0)),
            pl.BlockSpec((1, 1), lambda i: (0, 0)),
        ],
        out_shape=[
            jax.ShapeDtypeStruct((N_HOST, 64), jnp.float32),
            jax.ShapeDtypeStruct((N_HOST, 64), jnp.float32),
            jax.ShapeDtypeStruct((2, 64), jnp.float32),
            jax.ShapeDtypeStruct((1, 1), jnp.float32),
        ],
    )(acc_hh, ss_hh, acc_uh, ss_uh, kw, kb, r)


def _combine2_block(oh_ref, ou_ref, at_ref, w_ref, b_ref, a_ref, ys_ref, al_ref):
    i = pl.program_id(0)
    h = i % 2
    h2 = jnp.maximum(at_ref[0, 0] * oh_ref[...] + at_ref[0, 1] * ou_ref[...], 0.0)
    y = jnp.dot(h2, w_ref[...], preferred_element_type=jnp.float32) + b_ref[...]
    ys_ref[...] = jnp.where(h == 0, y[:, :32], y[:, 32:])
    al_ref[...] = jnp.dot(y, a_ref[...], preferred_element_type=jnp.float32)


def _combine2(o_hh, o_uh, attn, w2, b2, A2, block=1000):
    nb = N_HOST // block
    return pl.pallas_call(
        _combine2_block,
        grid=(2 * nb,),
        in_specs=[
            pl.BlockSpec((block, 64), lambda i: (i // 2, 0)),
            pl.BlockSpec((block, 64), lambda i: (i // 2, 0)),
            pl.BlockSpec((1, 2), lambda i: (0, 0)),
            pl.BlockSpec((64, 64), lambda i: (0, 0)),
            pl.BlockSpec((64,), lambda i: (0,)),
            pl.BlockSpec((64, 2), lambda i: (0, 0)),
        ],
        out_specs=[
            pl.BlockSpec((block, 32), lambda i: ((i % 2) * nb + i // 2, 0)),
            pl.BlockSpec((block, 2), lambda i: (i // 2, 0)),
        ],
        out_shape=[
            jax.ShapeDtypeStruct((2 * N_HOST, 32), jnp.float32),
            jax.ShapeDtypeStruct((N_HOST, 2), jnp.float32),
        ],
    )(o_hh, o_uh, attn, w2, b2, A2)


def _final_block(o2_ref, kw_ref, kb_ref, m_ref):
    i = pl.program_id(0)
    o2 = o2_ref[...]
    t = jnp.tanh(jnp.dot(o2, kw_ref[...],
                         preferred_element_type=jnp.float32) + kb_ref[...])

    @pl.when(i == 0)
    def _():
        m_ref[...] = jnp.zeros_like(m_ref)

    m_ref[...] = m_ref[...] + jnp.stack([o2.sum(axis=0), t.sum(axis=0)])


def _final_sums(o2, kw, kb, block=1000):
    nb = N_HOST // block
    return pl.pallas_call(
        _final_block,
        grid=(nb,),
        in_specs=[
            pl.BlockSpec((block, 64), lambda i: (i, 0)),
            pl.BlockSpec((64, 64), lambda i: (0, 0)),
            pl.BlockSpec((64,), lambda i: (0,)),
        ],
        out_specs=pl.BlockSpec((2, 64), lambda i: (0, 0)),
        out_shape=jax.ShapeDtypeStruct((2, 64), jnp.float32),
    )(o2, kw, kb)


def kernel(x_host, x_user, edge_index_host_auth_host, edge_index_user_authenticates_to_host,
           proj_host_w1, proj_host_b1, proj_user_w1, proj_user_b1,
           att_src_hh1, att_dst_hh1, att_src_uh1, att_dst_uh1, k_lin_w1, k_lin_b1, q1,
           proj_host_w2, proj_host_b2, proj_user_w2, proj_user_b2,
           att_src_hh2, att_dst_hh2, att_src_uh2, att_dst_uh2, k_lin_w2, k_lin_b2, q2,
           proj_w, proj_b):
    ei_hh = edge_index_host_auth_host
    ei_uh = edge_index_user_authenticates_to_host
    n = N_HOST

    E_HH_PAD = 802816
    E_UH_PAD = 409600
    src_hh, dst_hh = _pad_edges(ei_hh[0], ei_hh[1], N_HOST, E_HH_PAD)
    src_uh, dst_uh = _pad_edges(ei_uh[0], ei_uh[1], N_USER, E_UH_PAD)

    agg_hh = _make_sc_agg(E_HH_PAD, N_HOST, 2)
    agg_uh = _make_sc_agg(E_UH_PAD, N_USER, 2)
    agg_hh2 = _make_sc_agg(E_HH_PAD, N_HOST, 1)

    A_host1 = jnp.concatenate(
        [_att_mat(att_src_hh1), _att_mat(att_dst_hh1), _att_mat(att_dst_uh1)], axis=1)
    yhs, al_h = _project(x_host, proj_host_w1, proj_host_b1, A_host1)
    yus, als_uh = _project(x_user, proj_user_w1, proj_user_b1, _att_mat(att_src_uh1))

    acc_hh, ss_hh = agg_hh(src_hh, dst_hh, yhs,
                           jnp.concatenate([al_h[:, 0], al_h[:, 2]]),
                           jnp.concatenate([al_h[:, 1], al_h[:, 3]]),
                           _pad_dst_tab(al_h[:, 4], al_h[:, 6]),
                           _pad_dst_tab(al_h[:, 5], al_h[:, 7]))
    acc_uh, ss_uh = agg_uh(src_uh, dst_uh, yus,
                           jnp.concatenate([als_uh[:, 0], als_uh[:, 2]]),
                           jnp.concatenate([als_uh[:, 1], als_uh[:, 3]]),
                           _pad_dst_tab(al_h[:, 8], al_h[:, 10]),
                           _pad_dst_tab(al_h[:, 9], al_h[:, 11]))
    s4_hh = jnp.concatenate([ss_hh[0].reshape(NPAD, 2), ss_hh[1].reshape(NPAD, 2)],
                            axis=1)
    s4_uh = jnp.concatenate([ss_uh[0].reshape(NPAD, 2), ss_uh[1].reshape(NPAD, 2)],
                            axis=1)

    o_hh, o_uh, km, cov = _norm1(acc_hh, s4_hh, acc_uh, s4_uh, k_lin_w1, k_lin_b1)
    kmat = km / n
    score = (q1[None, :] * kmat).sum(-1)
    attn = jax.nn.softmax(score).reshape(1, 2)

    A_host2 = jnp.concatenate([_att_mat(att_src_hh2), _att_mat(att_dst_hh2)], axis=1)
    yh2s, al2 = _combine2(o_hh, o_uh, attn, proj_host_w2, proj_host_b2, A_host2)
    acc2, ss2 = agg_hh2(src_hh, dst_hh, yh2s,
                        jnp.concatenate([al2[:, 0], al2[:, 0]]),
                        jnp.concatenate([al2[:, 0], al2[:, 0]]),
                        _pad_dst_tab(al2[:, 1], al2[:, 1]),
                        _pad_dst_tab(al2[:, 1], al2[:, 1]))
    o2 = jnp.maximum(acc2[:N_HOST] / (ss2[0, :N_HOST, None] + 1e-16), 0.0)

    m = _final_sums(o2, k_lin_w2, k_lin_b2)
    frac = cov[0, 0] / n
    v1 = jax.nn.relu(proj_user_b2)
    kmat_hh2 = m[1] / n
    kmat_uh2 = frac * jnp.tanh(v1 @ k_lin_w2 + k_lin_b2) + (1 - frac) * jnp.tanh(k_lin_b2)
    score2 = jnp.stack([(q2 * kmat_hh2).sum(), (q2 * kmat_uh2).sum()])
    attn2 = jax.nn.softmax(score2)
    emb = attn2[0] * (m[0] / n) + attn2[1] * (frac * v1)
    return emb @ proj_w + proj_b
